# baseline (speedup 1.0000x reference)
import functools

import jax
import jax.numpy as jnp
from jax import lax
from jax.experimental import pallas as pl
from jax.experimental.pallas import tpu as pltpu
from jax.experimental.pallas import tpu_sc as plsc

N_NODES = 10000
NC = 2
NS = 16
NW = NC * NS
CHUNK = 128
N_PAD = 10240
ROWS_PER_TILE = N_PAD // NS
DUMMY = N_NODES
F = 128
KB = 2


@functools.lru_cache(maxsize=None)
def _make_segsum(n_chunks: int, width: int):
    mesh = plsc.VectorSubcoreMesh(core_axis_name="c", subcore_axis_name="s")

    assert n_chunks % KB == 0

    @functools.partial(
        pl.kernel, mesh=mesh,
        out_type=jax.ShapeDtypeStruct((NC * N_PAD, width), jnp.float32),
        scratch_types=(
            pltpu.VMEM((KB, 1, CHUNK), jnp.int32),
            pltpu.VMEM((KB, 1, CHUNK), jnp.int32),
            tuple(pltpu.VMEM((CHUNK, width), jnp.float32) for _ in range(KB)),
            pltpu.VMEM_SHARED((N_PAD, width), jnp.float32),
            pltpu.SemaphoreType.DMA,
            pltpu.SemaphoreType.DMA,
        ))
    def segsum(table, src3, dst3, zrow, out_sum,
               src_v, dst_v, rows, acc_sh, gsem, ssem):
        c = lax.axis_index("c")
        s = lax.axis_index("s")
        w = c * NS + s
        tile_row = s * ROWS_PER_TILE

        pltpu.sync_copy(zrow, rows[0])
        for j in range(ROWS_PER_TILE // CHUNK):
            pltpu.sync_copy(rows[0], acc_sh.at[pl.ds(tile_row + j * CHUNK, CHUNK)])
        plsc.subcore_barrier()

        base = w * n_chunks

        def outer(g, carry):
            off = base + g * KB
            pltpu.sync_copy(src3.at[pl.ds(off, KB)], src_v)
            pltpu.sync_copy(dst3.at[pl.ds(off, KB)], dst_v)
            for j in range(KB):
                pltpu.async_copy(table.at[src_v.at[j, 0]], rows[j], gsem)
            for j in range(KB):
                pltpu.make_async_copy(table.at[pl.ds(0, CHUNK)], rows[j], gsem).wait()
            for j in range(KB):
                pltpu.async_copy(rows[j], acc_sh.at[dst_v.at[j, 0]], ssem, add=True)
            for j in range(KB):
                pltpu.make_async_copy(table.at[pl.ds(0, CHUNK)], rows[j], ssem).wait()
            return carry
        lax.fori_loop(0, n_chunks // KB, outer, 0)
        plsc.subcore_barrier()

        for j in range(ROWS_PER_TILE // CHUNK):
            r = tile_row + j * CHUNK
            pltpu.sync_copy(acc_sh.at[pl.ds(r, CHUNK)], rows[0])
            pltpu.sync_copy(rows[0], out_sum.at[pl.ds(c * N_PAD + r, CHUNK)])

    return segsum


@functools.lru_cache(maxsize=None)
def _make_segdeg(n_chunks: int):
    mesh = plsc.VectorSubcoreMesh(core_axis_name="c", subcore_axis_name="s")

    @functools.partial(
        pl.kernel, mesh=mesh,
        out_type=jax.ShapeDtypeStruct((NW * ROWS_PER_TILE, 16), jnp.float32),
        compiler_params=pltpu.CompilerParams(needs_layout_passes=False),
        scratch_types=(
            pltpu.VMEM((CHUNK,), jnp.int32),
            pltpu.VMEM((ROWS_PER_TILE, 16), jnp.float32),
        ))
    def segdeg(dst, zdeg, out_deg, dst_v, acc):
        c = lax.axis_index("c")
        s = lax.axis_index("s")
        w = c * NS + s

        for j in range(ROWS_PER_TILE // CHUNK):
            pltpu.sync_copy(zdeg, acc.at[pl.ds(j * CHUNK, CHUNK)])

        base = w * n_chunks * CHUNK
        ones = jnp.ones((16,), jnp.float32)

        def body(i, carry):
            off = base + i * CHUNK
            pltpu.sync_copy(dst.at[pl.ds(off, CHUNK)], dst_v)
            for k in range(CHUNK // 16):
                iv = dst_v[pl.ds(k * 16, 16)]
                row = jax.lax.shift_right_logical(iv, 4)
                col = jnp.bitwise_and(iv, 15)
                plsc.addupdate_scatter(acc, (row, col), ones)
            return carry
        lax.fori_loop(0, n_chunks, body, 0)

        for j in range(ROWS_PER_TILE // CHUNK):
            pltpu.sync_copy(acc.at[pl.ds(j * CHUNK, CHUNK)],
                            out_deg.at[pl.ds(w * ROWS_PER_TILE + j * CHUNK, CHUNK)])

    return segdeg


_BN = 1000


def _dense1_body(p_ref, d0_ref, x_ref, wl_ref, wr_ref, b_ref, o_ref, d_ref):
    summed = p_ref[0] + p_ref[1]
    deg = jnp.maximum(jnp.sum(d0_ref[...], axis=1), 1.0)[:, None]
    mean = summed / deg
    h = (jnp.dot(mean, wl_ref[...], preferred_element_type=jnp.float32)
         + jnp.dot(x_ref[...], wr_ref[...], preferred_element_type=jnp.float32)
         + b_ref[...])
    o_ref[...] = jnp.maximum(h, 0.0)
    d_ref[...] = deg


def _dense1(P, D0, x, Wl, Wr, b):
    return pl.pallas_call(
        _dense1_body,
        grid=(N_NODES // _BN,),
        in_specs=[
            pl.BlockSpec((NC, _BN, F), lambda i: (0, i, 0)),
            pl.BlockSpec((_BN, NW), lambda i: (i, 0)),
            pl.BlockSpec((_BN, F), lambda i: (i, 0)),
            pl.BlockSpec(Wl.shape, lambda i: (0, 0)),
            pl.BlockSpec(Wr.shape, lambda i: (0, 0)),
            pl.BlockSpec(b.shape, lambda i: (0, 0)),
        ],
        out_specs=[
            pl.BlockSpec((_BN, 2 * F), lambda i: (i, 0)),
            pl.BlockSpec((_BN, 1), lambda i: (i, 0)),
        ],
        out_shape=[
            jax.ShapeDtypeStruct((N_NODES, 2 * F), jnp.float32),
            jax.ShapeDtypeStruct((N_NODES, 1), jnp.float32),
        ],
    )(P[:, :N_NODES], D0[:N_NODES], x, Wl, Wr, b)


def _dense2_body(pa_ref, pb_ref, d_ref, h_ref, w2l_ref, w2r_ref, b2_ref,
                 wh1_ref, bh1_ref, wh2_ref, bh2_ref, o_ref):
    summed = jnp.concatenate(
        [pa_ref[0] + pa_ref[1], pb_ref[0] + pb_ref[1]], axis=1)
    mean = summed / d_ref[...]
    h2 = (jnp.dot(mean, w2l_ref[...], preferred_element_type=jnp.float32)
          + jnp.dot(h_ref[...], w2r_ref[...], preferred_element_type=jnp.float32)
          + b2_ref[...])
    h2 = jnp.maximum(h2, 0.0)
    h3 = jnp.maximum(
        jnp.dot(h2, wh1_ref[...], preferred_element_type=jnp.float32)
        + bh1_ref[...], 0.0)
    o = jnp.dot(h3, wh2_ref[...], preferred_element_type=jnp.float32) + bh2_ref[...]
    o_ref[...] = jax.nn.sigmoid(o)


def _dense2(Pa, Pb, D, h, W2l, W2r, b2, Wh1, bh1, Wh2, bh2):
    return pl.pallas_call(
        _dense2_body,
        grid=(N_NODES // _BN,),
        in_specs=[
            pl.BlockSpec((NC, _BN, F), lambda i: (0, i, 0)),
            pl.BlockSpec((NC, _BN, F), lambda i: (0, i, 0)),
            pl.BlockSpec((_BN, 1), lambda i: (i, 0)),
            pl.BlockSpec((_BN, 2 * F), lambda i: (i, 0)),
            pl.BlockSpec(W2l.shape, lambda i: (0, 0)),
            pl.BlockSpec(W2r.shape, lambda i: (0, 0)),
            pl.BlockSpec(b2.shape, lambda i: (0, 0)),
            pl.BlockSpec(Wh1.shape, lambda i: (0, 0)),
            pl.BlockSpec(bh1.shape, lambda i: (0, 0)),
            pl.BlockSpec(Wh2.shape, lambda i: (0, 0)),
            pl.BlockSpec(bh2.shape, lambda i: (0, 0)),
        ],
        out_specs=pl.BlockSpec((_BN, 1), lambda i: (i, 0)),
        out_shape=jax.ShapeDtypeStruct((N_NODES, 1), jnp.float32),
    )(Pa[:, :N_NODES], Pb[:, :N_NODES], D, h,
      W2l, W2r, b2, Wh1, bh1, Wh2, bh2)


def kernel(x, edge_index, W1l, b1l, W1r, b1r, W2l, b2l, W2r, b2r,
           Wh1, bh1, Wh2, bh2):
    n_edges = edge_index.shape[1]
    n_chunks = -(-n_edges // (NW * CHUNK))
    n_chunks = -(-n_chunks // KB) * KB
    e_pad = NW * CHUNK * n_chunks

    src = edge_index[0].astype(jnp.int32)
    dst = edge_index[1].astype(jnp.int32)
    pad = e_pad - n_edges
    if pad:
        src = jnp.concatenate([src, jnp.zeros((pad,), jnp.int32)])
        dst = jnp.concatenate([dst, jnp.full((pad,), DUMMY, jnp.int32)])
    z128 = jnp.zeros((CHUNK, F), jnp.float32)
    zdeg = jnp.zeros((CHUNK, 16), jnp.float32)

    seg = _make_segsum(n_chunks, F)
    segdeg = _make_segdeg(n_chunks)

    src3 = src.reshape(-1, 1, CHUNK)
    dst3 = dst.reshape(-1, 1, CHUNK)

    D0 = segdeg(dst, zdeg).reshape(NW, N_PAD).T
    P1 = seg(x, src3, dst3, z128).reshape(NC, N_PAD, F)
    h, D = _dense1(P1, D0, x, W1l, W1r, (b1l + b1r).reshape(1, -1))
    Pa = seg(h[:, :F], src3, dst3, z128).reshape(NC, N_PAD, F)
    Pb = seg(h[:, F:], src3, dst3, z128).reshape(NC, N_PAD, F)
    out = _dense2(Pa, Pb, D, h, W2l, W2r, (b2l + b2r).reshape(1, -1),
                  Wh1, bh1.reshape(1, -1), Wh2, bh2.reshape(1, -1))
    return out


# speedup vs baseline: 1.3476x; 1.3476x over previous
import functools

import jax
import jax.numpy as jnp
from jax import lax
from jax.experimental import pallas as pl
from jax.experimental.pallas import tpu as pltpu
from jax.experimental.pallas import tpu_sc as plsc

N_NODES = 10000
NC = 2
NS = 16
NW = NC * NS
CHUNK = 128
N_PAD = 10240
ROWS_PER_TILE = N_PAD // NS
DUMMY = N_NODES
F = 128


@functools.lru_cache(maxsize=None)
def _make_segsum(n_chunks: int):
    mesh = plsc.VectorSubcoreMesh(core_axis_name="c", subcore_axis_name="s")

    @functools.partial(
        pl.kernel, mesh=mesh,
        out_type=jax.ShapeDtypeStruct((NC * N_PAD, F), jnp.float32),
        scratch_types=(
            pltpu.VMEM((CHUNK,), jnp.int32),
            pltpu.VMEM((CHUNK,), jnp.int32),
            pltpu.VMEM((CHUNK, F), jnp.float32),
            pltpu.VMEM_SHARED((N_PAD, F), jnp.float32),
            pltpu.SemaphoreType.DMA,
        ))
    def segsum(table, src, dst, zrow, out_sum, src_v, dst_v, rows_v, acc_sh, sem):
        c = lax.axis_index("c")
        s = lax.axis_index("s")
        w = c * NS + s
        tile_row = s * ROWS_PER_TILE

        pltpu.sync_copy(zrow, rows_v)
        for j in range(ROWS_PER_TILE // CHUNK):
            pltpu.sync_copy(rows_v, acc_sh.at[pl.ds(tile_row + j * CHUNK, CHUNK)])
        plsc.subcore_barrier()

        base = w * n_chunks * CHUNK

        def body(i, carry):
            off = base + i * CHUNK
            pltpu.sync_copy(src.at[pl.ds(off, CHUNK)], src_v)
            pltpu.sync_copy(dst.at[pl.ds(off, CHUNK)], dst_v)
            pltpu.async_copy(table.at[src_v], rows_v, sem).wait()
            pltpu.sync_copy(rows_v, acc_sh.at[dst_v], add=True)
            return carry
        lax.fori_loop(0, n_chunks, body, 0)
        plsc.subcore_barrier()

        for j in range(ROWS_PER_TILE // CHUNK):
            r = tile_row + j * CHUNK
            pltpu.sync_copy(acc_sh.at[pl.ds(r, CHUNK)], rows_v)
            pltpu.sync_copy(rows_v, out_sum.at[pl.ds(c * N_PAD + r, CHUNK)])

    return segsum


@functools.lru_cache(maxsize=None)
def _make_segdeg(n_chunks: int):
    mesh = plsc.VectorSubcoreMesh(core_axis_name="c", subcore_axis_name="s")

    @functools.partial(
        pl.kernel, mesh=mesh,
        out_type=jax.ShapeDtypeStruct((NW * ROWS_PER_TILE, 16), jnp.float32),
        compiler_params=pltpu.CompilerParams(needs_layout_passes=False),
        scratch_types=(
            pltpu.VMEM((CHUNK,), jnp.int32),
            pltpu.VMEM((ROWS_PER_TILE, 16), jnp.float32),
        ))
    def segdeg(dst, zdeg, out_deg, dst_v, acc):
        c = lax.axis_index("c")
        s = lax.axis_index("s")
        w = c * NS + s

        for j in range(ROWS_PER_TILE // CHUNK):
            pltpu.sync_copy(zdeg, acc.at[pl.ds(j * CHUNK, CHUNK)])

        base = w * n_chunks * CHUNK
        ones = jnp.ones((16,), jnp.float32)

        def body(i, carry):
            off = base + i * CHUNK
            pltpu.sync_copy(dst.at[pl.ds(off, CHUNK)], dst_v)
            for k in range(CHUNK // 16):
                iv = dst_v[pl.ds(k * 16, 16)]
                row = jax.lax.shift_right_logical(iv, 4)
                col = jnp.bitwise_and(iv, 15)
                plsc.addupdate_scatter(acc, (row, col), ones)
            return carry
        lax.fori_loop(0, n_chunks, body, 0)

        for j in range(ROWS_PER_TILE // CHUNK):
            pltpu.sync_copy(acc.at[pl.ds(j * CHUNK, CHUNK)],
                            out_deg.at[pl.ds(w * ROWS_PER_TILE + j * CHUNK, CHUNK)])

    return segdeg


_BN = 1000


def _dense1_body(p_ref, d0_ref, x_ref, wl_ref, wr_ref, b_ref, o_ref, d_ref):
    summed = p_ref[0] + p_ref[1]
    deg = jnp.maximum(jnp.sum(d0_ref[...], axis=1), 1.0)[:, None]
    mean = summed / deg
    h = (jnp.dot(mean, wl_ref[...], preferred_element_type=jnp.float32)
         + jnp.dot(x_ref[...], wr_ref[...], preferred_element_type=jnp.float32)
         + b_ref[...])
    o_ref[...] = jnp.maximum(h, 0.0)
    d_ref[...] = deg


def _dense1(P, D0, x, Wl, Wr, b):
    return pl.pallas_call(
        _dense1_body,
        grid=(N_NODES // _BN,),
        in_specs=[
            pl.BlockSpec((NC, _BN, F), lambda i: (0, i, 0)),
            pl.BlockSpec((_BN, NW), lambda i: (i, 0)),
            pl.BlockSpec((_BN, F), lambda i: (i, 0)),
            pl.BlockSpec(Wl.shape, lambda i: (0, 0)),
            pl.BlockSpec(Wr.shape, lambda i: (0, 0)),
            pl.BlockSpec(b.shape, lambda i: (0, 0)),
        ],
        out_specs=[
            pl.BlockSpec((_BN, 2 * F), lambda i: (i, 0)),
            pl.BlockSpec((_BN, 1), lambda i: (i, 0)),
        ],
        out_shape=[
            jax.ShapeDtypeStruct((N_NODES, 2 * F), jnp.float32),
            jax.ShapeDtypeStruct((N_NODES, 1), jnp.float32),
        ],
    )(P[:, :N_NODES], D0[:N_NODES], x, Wl, Wr, b)


def _dense2_body(pa_ref, pb_ref, d_ref, h_ref, w2l_ref, w2r_ref, b2_ref,
                 wh1_ref, bh1_ref, wh2_ref, bh2_ref, o_ref):
    summed = jnp.concatenate(
        [pa_ref[0] + pa_ref[1], pb_ref[0] + pb_ref[1]], axis=1)
    mean = summed / d_ref[...]
    h2 = (jnp.dot(mean, w2l_ref[...], preferred_element_type=jnp.float32)
          + jnp.dot(h_ref[...], w2r_ref[...], preferred_element_type=jnp.float32)
          + b2_ref[...])
    h2 = jnp.maximum(h2, 0.0)
    h3 = jnp.maximum(
        jnp.dot(h2, wh1_ref[...], preferred_element_type=jnp.float32)
        + bh1_ref[...], 0.0)
    o = jnp.dot(h3, wh2_ref[...], preferred_element_type=jnp.float32) + bh2_ref[...]
    o_ref[...] = jax.nn.sigmoid(o)


def _dense2(Pa, Pb, D, h, W2l, W2r, b2, Wh1, bh1, Wh2, bh2):
    return pl.pallas_call(
        _dense2_body,
        grid=(N_NODES // _BN,),
        in_specs=[
            pl.BlockSpec((NC, _BN, F), lambda i: (0, i, 0)),
            pl.BlockSpec((NC, _BN, F), lambda i: (0, i, 0)),
            pl.BlockSpec((_BN, 1), lambda i: (i, 0)),
            pl.BlockSpec((_BN, 2 * F), lambda i: (i, 0)),
            pl.BlockSpec(W2l.shape, lambda i: (0, 0)),
            pl.BlockSpec(W2r.shape, lambda i: (0, 0)),
            pl.BlockSpec(b2.shape, lambda i: (0, 0)),
            pl.BlockSpec(Wh1.shape, lambda i: (0, 0)),
            pl.BlockSpec(bh1.shape, lambda i: (0, 0)),
            pl.BlockSpec(Wh2.shape, lambda i: (0, 0)),
            pl.BlockSpec(bh2.shape, lambda i: (0, 0)),
        ],
        out_specs=pl.BlockSpec((_BN, 1), lambda i: (i, 0)),
        out_shape=jax.ShapeDtypeStruct((N_NODES, 1), jnp.float32),
    )(Pa[:, :N_NODES], Pb[:, :N_NODES], D, h,
      W2l, W2r, b2, Wh1, bh1, Wh2, bh2)


def kernel(x, edge_index, W1l, b1l, W1r, b1r, W2l, b2l, W2r, b2r,
           Wh1, bh1, Wh2, bh2):
    n_edges = edge_index.shape[1]
    n_chunks = -(-n_edges // (NW * CHUNK))
    e_pad = NW * CHUNK * n_chunks

    src = edge_index[0].astype(jnp.int32)
    dst = edge_index[1].astype(jnp.int32)
    pad = e_pad - n_edges
    if pad:
        src = jnp.concatenate([src, jnp.zeros((pad,), jnp.int32)])
        dst = jnp.concatenate([dst, jnp.full((pad,), DUMMY, jnp.int32)])
    z128 = jnp.zeros((CHUNK, F), jnp.float32)
    zdeg = jnp.zeros((CHUNK, 16), jnp.float32)

    seg = _make_segsum(n_chunks)
    segdeg = _make_segdeg(n_chunks)

    D0 = segdeg(dst, zdeg).reshape(NW, N_PAD).T
    P1 = seg(x, src, dst, z128).reshape(NC, N_PAD, F)
    h, D = _dense1(P1, D0, x, W1l, W1r, (b1l + b1r).reshape(1, -1))
    Pa = seg(h[:, :F], src, dst, z128).reshape(NC, N_PAD, F)
    Pb = seg(h[:, F:], src, dst, z128).reshape(NC, N_PAD, F)
    out = _dense2(Pa, Pb, D, h, W2l, W2r, (b2l + b2r).reshape(1, -1),
                  Wh1, bh1.reshape(1, -1), Wh2, bh2.reshape(1, -1))
    return out
